# baseline (speedup 1.0000x reference)
import functools

import jax
import jax.numpy as jnp
from jax import lax
from jax.experimental import pallas as pl
from jax.experimental.pallas import tpu as pltpu
from jax.experimental.pallas import tpu_sc as plsc

B, T = 8, 2048
K, CDIM, OUT_DIM = 1024, 1024, 768
N_TOK = B * T
HALF = OUT_DIM // 2
NSPLIT = 4
B_SPLIT = B // NSPLIT
TOK_SPLIT = N_TOK // NSPLIT

def _proj_body(cent_ref, w_ref, b_ref, p_ref):
    p = lax.dot_general(
        cent_ref[...].astype(jnp.bfloat16), w_ref[...].astype(jnp.bfloat16),
        dimension_numbers=(((1,), (1,)), ((), ())),
        preferred_element_type=jnp.float32,
    ) + b_ref[...]
    lo = lax.bitcast_convert_type(p[:, :HALF].astype(jnp.bfloat16), jnp.int16)
    hi = lax.bitcast_convert_type(p[:, HALF:].astype(jnp.bfloat16), jnp.int16)
    p_ref[...] = (lo.astype(jnp.int32) & 0xFFFF) | (hi.astype(jnp.int32) << 16)


def _project(centroids, W, b):
    return pl.pallas_call(
        _proj_body,
        out_shape=jax.ShapeDtypeStruct((K, HALF), jnp.int32),
    )(centroids, W, b.reshape(1, OUT_DIM))


_NC, _NS = 2, 16
_NW = _NC * _NS
_ROWS_PER_W = TOK_SPLIT // _NW
_CHUNK = 64
_NCHUNK = _ROWS_PER_W // _CHUNK


def _gather_kernel(table_hbm, idx_hbm, out_hbm, idx_v, *scratch):
    bufs = scratch[:_NCHUNK]
    rsems = scratch[_NCHUNK:2 * _NCHUNK]
    wsems = scratch[2 * _NCHUNK:]
    wid = lax.axis_index("s") * _NC + lax.axis_index("c")
    base = wid * _ROWS_PER_W
    pltpu.sync_copy(idx_hbm.at[pl.ds(base, _ROWS_PER_W)], idx_v)
    reads = [
        pltpu.async_copy(
            table_hbm.at[idx_v.at[pl.ds(k * _CHUNK, _CHUNK)]],
            bufs[k], rsems[k])
        for k in range(_NCHUNK)
    ]
    writes = []
    for k in range(_NCHUNK):
        reads[k].wait()
        writes.append(pltpu.async_copy(
            bufs[k], out_hbm.at[pl.ds(base + k * _CHUNK, _CHUNK)], wsems[k]))
    for w in writes:
        w.wait()


def _gather(table, idx):
    mesh = plsc.VectorSubcoreMesh(core_axis_name="c", subcore_axis_name="s")
    return pl.kernel(
        _gather_kernel,
        mesh=mesh,
        out_type=jax.ShapeDtypeStruct((TOK_SPLIT, HALF), jnp.int32),
        scratch_types=(
            [pltpu.VMEM((_ROWS_PER_W,), jnp.int32)]
            + [pltpu.VMEM((_CHUNK, HALF), jnp.int32)] * _NCHUNK
            + [pltpu.SemaphoreType.DMA] * (2 * _NCHUNK)
        ),
    )(table, idx)


def _unpack(words):
    lo16 = ((words << 16) >> 16).astype(jnp.int16)
    return lax.bitcast_convert_type(lo16, jnp.bfloat16).astype(jnp.float32)


def _tpose_first_body(g_ref, o_ref):
    gt = g_ref[...].T
    o_ref[0, :HALF] = _unpack(gt)
    o_ref[0, HALF:] = _unpack(gt >> 16)


def _tpose_chain_body(prev_ref, g_ref, o_ref):
    del prev_ref
    gt = g_ref[...].T
    o_ref[0, :HALF] = _unpack(gt)
    o_ref[0, HALF:] = _unpack(gt >> 16)


_OUT_SHAPE = jax.ShapeDtypeStruct((B, OUT_DIM, T), jnp.float32)


def _transpose_first(g):
    return pl.pallas_call(
        _tpose_first_body,
        grid=(B_SPLIT,),
        in_specs=[pl.BlockSpec((T, HALF), lambda i: (i, 0))],
        out_specs=pl.BlockSpec((1, OUT_DIM, T), lambda i: (i, 0, 0)),
        out_shape=_OUT_SHAPE,
        compiler_params=pltpu.CompilerParams(
            dimension_semantics=("parallel",)),
    )(g)


def _transpose_chain(prev, g, b_off):
    return pl.pallas_call(
        _tpose_chain_body,
        grid=(B_SPLIT,),
        in_specs=[
            pl.BlockSpec(memory_space=pl.ANY),
            pl.BlockSpec((T, HALF), lambda i: (i, 0)),
        ],
        out_specs=pl.BlockSpec(
            (1, OUT_DIM, T), lambda i, o=b_off: (i + o, 0, 0)),
        out_shape=_OUT_SHAPE,
        input_output_aliases={0: 0},
        compiler_params=pltpu.CompilerParams(
            dimension_semantics=("parallel",)),
    )(prev, g)


def kernel(c, centroids, W, b):
    proj_table = _project(centroids, W, b)
    idx = c.reshape(-1).astype(jnp.int32)
    gs = [_gather(proj_table, idx[k * TOK_SPLIT:(k + 1) * TOK_SPLIT])
          for k in range(NSPLIT)]
    out = _transpose_first(gs[0])
    for k in range(1, NSPLIT):
        out = _transpose_chain(out, gs[k], k * B_SPLIT)
    return out


# speedup vs baseline: 1.0992x; 1.0992x over previous
import functools

import jax
import jax.numpy as jnp
from jax import lax
from jax.experimental import pallas as pl
from jax.experimental.pallas import tpu as pltpu
from jax.experimental.pallas import tpu_sc as plsc

B, T = 8, 2048
K, CDIM, OUT_DIM = 1024, 1024, 768
N_TOK = B * T
HALF = OUT_DIM // 2
NSPLIT = 2
B_SPLIT = B // NSPLIT
TOK_SPLIT = N_TOK // NSPLIT

def _proj_body(cent_ref, w_ref, b_ref, p_ref):
    p = lax.dot_general(
        cent_ref[...].astype(jnp.bfloat16), w_ref[...].astype(jnp.bfloat16),
        dimension_numbers=(((1,), (1,)), ((), ())),
        preferred_element_type=jnp.float32,
    ) + b_ref[...]
    lo = lax.bitcast_convert_type(p[:, :HALF].astype(jnp.bfloat16), jnp.int16)
    hi = lax.bitcast_convert_type(p[:, HALF:].astype(jnp.bfloat16), jnp.int16)
    p_ref[...] = (lo.astype(jnp.int32) & 0xFFFF) | (hi.astype(jnp.int32) << 16)


def _project(centroids, W, b):
    return pl.pallas_call(
        _proj_body,
        out_shape=jax.ShapeDtypeStruct((K, HALF), jnp.int32),
    )(centroids, W, b.reshape(1, OUT_DIM))


_NC, _NS = 2, 16
_NW = _NC * _NS
_ROWS_PER_W = TOK_SPLIT // _NW
_CHUNK = 64
_NCHUNK = _ROWS_PER_W // _CHUNK


def _gather_kernel(table_hbm, idx_hbm, out_hbm, idx_v, *scratch):
    bufs = scratch[:_NCHUNK]
    rsems = scratch[_NCHUNK:2 * _NCHUNK]
    wsems = scratch[2 * _NCHUNK:]
    wid = lax.axis_index("s") * _NC + lax.axis_index("c")
    base = wid * _ROWS_PER_W
    pltpu.sync_copy(idx_hbm.at[pl.ds(base, _ROWS_PER_W)], idx_v)
    reads = [
        pltpu.async_copy(
            table_hbm.at[idx_v.at[pl.ds(k * _CHUNK, _CHUNK)]],
            bufs[k], rsems[k])
        for k in range(_NCHUNK)
    ]
    writes = []
    for k in range(_NCHUNK):
        reads[k].wait()
        writes.append(pltpu.async_copy(
            bufs[k], out_hbm.at[pl.ds(base + k * _CHUNK, _CHUNK)], wsems[k]))
    for w in writes:
        w.wait()


def _gather(table, idx):
    mesh = plsc.VectorSubcoreMesh(core_axis_name="c", subcore_axis_name="s")
    return pl.kernel(
        _gather_kernel,
        mesh=mesh,
        out_type=jax.ShapeDtypeStruct((TOK_SPLIT, HALF), jnp.int32),
        scratch_types=(
            [pltpu.VMEM((_ROWS_PER_W,), jnp.int32)]
            + [pltpu.VMEM((_CHUNK, HALF), jnp.int32)] * _NCHUNK
            + [pltpu.SemaphoreType.DMA] * (2 * _NCHUNK)
        ),
    )(table, idx)


def _unpack(words):
    lo16 = ((words << 16) >> 16).astype(jnp.int16)
    return lax.bitcast_convert_type(lo16, jnp.bfloat16).astype(jnp.float32)


def _tpose_first_body(g_ref, o_ref):
    gt = g_ref[...].T
    o_ref[0, :HALF] = _unpack(gt)
    o_ref[0, HALF:] = _unpack(gt >> 16)


def _tpose_chain_body(prev_ref, g_ref, o_ref):
    del prev_ref
    gt = g_ref[...].T
    o_ref[0, :HALF] = _unpack(gt)
    o_ref[0, HALF:] = _unpack(gt >> 16)


_OUT_SHAPE = jax.ShapeDtypeStruct((B, OUT_DIM, T), jnp.float32)


def _transpose_first(g):
    return pl.pallas_call(
        _tpose_first_body,
        grid=(B_SPLIT,),
        in_specs=[pl.BlockSpec((T, HALF), lambda i: (i, 0))],
        out_specs=pl.BlockSpec((1, OUT_DIM, T), lambda i: (i, 0, 0)),
        out_shape=_OUT_SHAPE,
        compiler_params=pltpu.CompilerParams(
            dimension_semantics=("parallel",)),
    )(g)


def _transpose_chain(prev, g, b_off):
    return pl.pallas_call(
        _tpose_chain_body,
        grid=(B_SPLIT,),
        in_specs=[
            pl.BlockSpec(memory_space=pl.ANY),
            pl.BlockSpec((T, HALF), lambda i: (i, 0)),
        ],
        out_specs=pl.BlockSpec(
            (1, OUT_DIM, T), lambda i, o=b_off: (i + o, 0, 0)),
        out_shape=_OUT_SHAPE,
        input_output_aliases={0: 0},
        compiler_params=pltpu.CompilerParams(
            dimension_semantics=("parallel",)),
    )(prev, g)


def kernel(c, centroids, W, b):
    proj_table = _project(centroids, W, b)
    idx = c.reshape(-1).astype(jnp.int32)
    gs = [_gather(proj_table, idx[k * TOK_SPLIT:(k + 1) * TOK_SPLIT])
          for k in range(NSPLIT)]
    out = _transpose_first(gs[0])
    for k in range(1, NSPLIT):
        out = _transpose_chain(out, gs[k], k * B_SPLIT)
    return out


# speedup vs baseline: 1.1041x; 1.0045x over previous
import functools

import jax
import jax.numpy as jnp
from jax import lax
from jax.experimental import pallas as pl
from jax.experimental.pallas import tpu as pltpu
from jax.experimental.pallas import tpu_sc as plsc

B, T = 8, 2048
K, CDIM, OUT_DIM = 1024, 1024, 768
N_TOK = B * T
HALF = OUT_DIM // 2
NSPLIT = 2
B_SPLIT = B // NSPLIT
TOK_SPLIT = N_TOK // NSPLIT

def _proj_body(cent_ref, w_ref, b_ref, p_ref):
    p = lax.dot_general(
        cent_ref[...].astype(jnp.bfloat16), w_ref[...].astype(jnp.bfloat16),
        dimension_numbers=(((1,), (1,)), ((), ())),
        preferred_element_type=jnp.float32,
    ) + b_ref[...]
    lo = lax.bitcast_convert_type(p[:, :HALF].astype(jnp.bfloat16), jnp.int16)
    hi = lax.bitcast_convert_type(p[:, HALF:].astype(jnp.bfloat16), jnp.int16)
    p_ref[...] = (lo.astype(jnp.int32) & 0xFFFF) | (hi.astype(jnp.int32) << 16)


def _project(centroids, W, b):
    return pl.pallas_call(
        _proj_body,
        out_shape=jax.ShapeDtypeStruct((K, HALF), jnp.int32),
    )(centroids, W, b.reshape(1, OUT_DIM))


_NC, _NS = 2, 16
_NW = _NC * _NS
_ROWS_PER_W = TOK_SPLIT // _NW
_CHUNK = 64
_NCHUNK = _ROWS_PER_W // _CHUNK


def _gather_kernel(koff, table_hbm, idx_hbm, out_hbm, idx_v, *scratch):
    bufs = scratch[:_NCHUNK]
    rsems = scratch[_NCHUNK:2 * _NCHUNK]
    wsems = scratch[2 * _NCHUNK:]
    wid = lax.axis_index("s") * _NC + lax.axis_index("c")
    base = wid * _ROWS_PER_W
    flat = koff + base
    pltpu.sync_copy(idx_hbm.at[flat // T, pl.ds(flat % T, _ROWS_PER_W)], idx_v)
    reads = [
        pltpu.async_copy(
            table_hbm.at[idx_v.at[pl.ds(k * _CHUNK, _CHUNK)]],
            bufs[k], rsems[k])
        for k in range(_NCHUNK)
    ]
    writes = []
    for k in range(_NCHUNK):
        reads[k].wait()
        writes.append(pltpu.async_copy(
            bufs[k], out_hbm.at[pl.ds(base + k * _CHUNK, _CHUNK)], wsems[k]))
    for w in writes:
        w.wait()


def _gather(table, codes, k):
    mesh = plsc.VectorSubcoreMesh(core_axis_name="c", subcore_axis_name="s")
    return pl.kernel(
        functools.partial(_gather_kernel, k * TOK_SPLIT),
        mesh=mesh,
        out_type=jax.ShapeDtypeStruct((TOK_SPLIT, HALF), jnp.int32),
        scratch_types=(
            [pltpu.VMEM((_ROWS_PER_W,), jnp.int32)]
            + [pltpu.VMEM((_CHUNK, HALF), jnp.int32)] * _NCHUNK
            + [pltpu.SemaphoreType.DMA] * (2 * _NCHUNK)
        ),
    )(table, codes)


def _unpack(words):
    lo16 = ((words << 16) >> 16).astype(jnp.int16)
    return lax.bitcast_convert_type(lo16, jnp.bfloat16).astype(jnp.float32)


def _tpose_first_body(g_ref, o_ref):
    gt = g_ref[...].T
    o_ref[0, :HALF] = _unpack(gt)
    o_ref[0, HALF:] = _unpack(gt >> 16)


def _tpose_chain_body(prev_ref, g_ref, o_ref):
    del prev_ref
    gt = g_ref[...].T
    o_ref[0, :HALF] = _unpack(gt)
    o_ref[0, HALF:] = _unpack(gt >> 16)


_OUT_SHAPE = jax.ShapeDtypeStruct((B, OUT_DIM, T), jnp.float32)


def _transpose_first(g):
    return pl.pallas_call(
        _tpose_first_body,
        grid=(B_SPLIT,),
        in_specs=[pl.BlockSpec((T, HALF), lambda i: (i, 0))],
        out_specs=pl.BlockSpec((1, OUT_DIM, T), lambda i: (i, 0, 0)),
        out_shape=_OUT_SHAPE,
        compiler_params=pltpu.CompilerParams(
            dimension_semantics=("parallel",)),
    )(g)


def _transpose_chain(prev, g, b_off):
    return pl.pallas_call(
        _tpose_chain_body,
        grid=(B_SPLIT,),
        in_specs=[
            pl.BlockSpec(memory_space=pl.ANY),
            pl.BlockSpec((T, HALF), lambda i: (i, 0)),
        ],
        out_specs=pl.BlockSpec(
            (1, OUT_DIM, T), lambda i, o=b_off: (i + o, 0, 0)),
        out_shape=_OUT_SHAPE,
        input_output_aliases={0: 0},
        compiler_params=pltpu.CompilerParams(
            dimension_semantics=("parallel",)),
    )(prev, g)


def kernel(c, centroids, W, b):
    proj_table = _project(centroids, W, b)
    codes = c.astype(jnp.int32)
    gs = [_gather(proj_table, codes, k) for k in range(NSPLIT)]
    out = _transpose_first(gs[0])
    for k in range(1, NSPLIT):
        out = _transpose_chain(out, gs[k], k * B_SPLIT)
    return out


# speedup vs baseline: 1.4457x; 1.3093x over previous
import functools

import jax
import jax.numpy as jnp
from jax import lax
from jax.experimental import pallas as pl
from jax.experimental.pallas import tpu as pltpu
from jax.experimental.pallas import tpu_sc as plsc

B, T = 8, 2048
K, CDIM, OUT_DIM = 1024, 1024, 768
N_TOK = B * T
HALF = OUT_DIM // 2
B_SC = 4
B_TC = B - B_SC
TOK_SC = B_SC * T

def _proj_body(cent_ref, w_ref, b_ref, packed_ref, p16_ref):
    p = lax.dot_general(
        cent_ref[...].astype(jnp.bfloat16), w_ref[...].astype(jnp.bfloat16),
        dimension_numbers=(((1,), (1,)), ((), ())),
        preferred_element_type=jnp.float32,
    ) + b_ref[...]
    pb = p.astype(jnp.bfloat16)
    lo = lax.bitcast_convert_type(pb[:, :HALF], jnp.int16)
    hi = lax.bitcast_convert_type(pb[:, HALF:], jnp.int16)
    packed_ref[...] = (lo.astype(jnp.int32) & 0xFFFF) | (hi.astype(jnp.int32) << 16)
    p16_ref[...] = pb


def _project(centroids, W, b):
    return pl.pallas_call(
        _proj_body,
        out_shape=[
            jax.ShapeDtypeStruct((K, HALF), jnp.int32),
            jax.ShapeDtypeStruct((K, OUT_DIM), jnp.bfloat16),
        ],
    )(centroids, W, b.reshape(1, OUT_DIM))


_NC, _NS = 2, 16
_NW = _NC * _NS
_ROWS_PER_W = TOK_SC // _NW
_CHUNK = 64
_NCHUNK = _ROWS_PER_W // _CHUNK


def _gather_kernel(table_hbm, idx_hbm, out_hbm, idx_v, *scratch):
    bufs = scratch[:_NCHUNK]
    rsems = scratch[_NCHUNK:2 * _NCHUNK]
    wsems = scratch[2 * _NCHUNK:]
    wid = lax.axis_index("s") * _NC + lax.axis_index("c")
    base = wid * _ROWS_PER_W
    pltpu.sync_copy(idx_hbm.at[base // T, pl.ds(base % T, _ROWS_PER_W)], idx_v)
    reads = [
        pltpu.async_copy(
            table_hbm.at[idx_v.at[pl.ds(k * _CHUNK, _CHUNK)]],
            bufs[k], rsems[k])
        for k in range(_NCHUNK)
    ]
    writes = []
    for k in range(_NCHUNK):
        reads[k].wait()
        writes.append(pltpu.async_copy(
            bufs[k], out_hbm.at[pl.ds(base + k * _CHUNK, _CHUNK)], wsems[k]))
    for w in writes:
        w.wait()


def _gather(table, codes):
    mesh = plsc.VectorSubcoreMesh(core_axis_name="c", subcore_axis_name="s")
    return pl.kernel(
        _gather_kernel,
        mesh=mesh,
        out_type=jax.ShapeDtypeStruct((TOK_SC, HALF), jnp.int32),
        scratch_types=(
            [pltpu.VMEM((_ROWS_PER_W,), jnp.int32)]
            + [pltpu.VMEM((_CHUNK, HALF), jnp.int32)] * _NCHUNK
            + [pltpu.SemaphoreType.DMA] * (2 * _NCHUNK)
        ),
    )(table, codes)


_OUT_SHAPE = jax.ShapeDtypeStruct((B, OUT_DIM, T), jnp.float32)


def _onehot_body(codes_ref, p_ref, o_ref):
    cb = codes_ref[0, 0, :]
    iota = lax.broadcasted_iota(jnp.int32, (K, T), 0)
    oh = (iota == cb[None, :]).astype(jnp.bfloat16)
    o_ref[0] = lax.dot_general(
        p_ref[...], oh,
        dimension_numbers=(((0,), (0,)), ((), ())),
        preferred_element_type=jnp.float32,
    )


def _onehot(codes, p16):
    return pl.pallas_call(
        _onehot_body,
        grid=(B_TC,),
        in_specs=[
            pl.BlockSpec((1, 1, T), lambda i: (i + B_SC, 0, 0)),
            pl.BlockSpec((K, OUT_DIM), lambda i: (0, 0)),
        ],
        out_specs=pl.BlockSpec((1, OUT_DIM, T), lambda i: (i + B_SC, 0, 0)),
        out_shape=_OUT_SHAPE,
        compiler_params=pltpu.CompilerParams(
            dimension_semantics=("arbitrary",)),
    )(codes.reshape(B, 1, T), p16)


def _unpack(words):
    lo16 = ((words << 16) >> 16).astype(jnp.int16)
    return lax.bitcast_convert_type(lo16, jnp.bfloat16).astype(jnp.float32)


def _tpose_body(prev_ref, g_ref, o_ref):
    del prev_ref
    gt = g_ref[...].T
    o_ref[0, :HALF] = _unpack(gt)
    o_ref[0, HALF:] = _unpack(gt >> 16)


def _transpose(prev, g):
    return pl.pallas_call(
        _tpose_body,
        grid=(B_SC,),
        in_specs=[
            pl.BlockSpec(memory_space=pl.ANY),
            pl.BlockSpec((T, HALF), lambda i: (i, 0)),
        ],
        out_specs=pl.BlockSpec((1, OUT_DIM, T), lambda i: (i, 0, 0)),
        out_shape=_OUT_SHAPE,
        input_output_aliases={0: 0},
        compiler_params=pltpu.CompilerParams(
            dimension_semantics=("parallel",)),
    )(prev, g)


def kernel(c, centroids, W, b):
    packed, p16 = _project(centroids, W, b)
    codes = c.astype(jnp.int32)
    g = _gather(packed, codes)
    out = _onehot(codes, p16)
    return _transpose(out, g)


# speedup vs baseline: 1.4814x; 1.0247x over previous
import functools

import jax
import jax.numpy as jnp
from jax import lax
from jax.experimental import pallas as pl
from jax.experimental.pallas import tpu as pltpu
from jax.experimental.pallas import tpu_sc as plsc

B, T = 8, 2048
K, CDIM, OUT_DIM = 1024, 1024, 768
N_TOK = B * T
HALF = OUT_DIM // 2
B_SC = 4
B_TC = B - B_SC
TOK_SC = B_SC * T

def _proj_body(cent_ref, w_ref, b_ref, packed_ref, p16_ref):
    p = lax.dot_general(
        cent_ref[...].astype(jnp.bfloat16), w_ref[...].astype(jnp.bfloat16),
        dimension_numbers=(((1,), (1,)), ((), ())),
        preferred_element_type=jnp.float32,
    ) + b_ref[...]
    pb = p.astype(jnp.bfloat16)
    lo = lax.bitcast_convert_type(pb[:, :HALF], jnp.int16)
    hi = lax.bitcast_convert_type(pb[:, HALF:], jnp.int16)
    packed_ref[...] = (lo.astype(jnp.int32) & 0xFFFF) | (hi.astype(jnp.int32) << 16)
    p16_ref[...] = pb


def _project(centroids, W, b):
    return pl.pallas_call(
        _proj_body,
        out_shape=[
            jax.ShapeDtypeStruct((K, HALF), jnp.int32),
            jax.ShapeDtypeStruct((K, OUT_DIM), jnp.bfloat16),
        ],
    )(centroids, W, b.reshape(1, OUT_DIM))


_NC, _NS = 2, 16
_NW = _NC * _NS
_ROWS_PER_W = TOK_SC // _NW
_CHUNK = 64
_NCHUNK = _ROWS_PER_W // _CHUNK


def _gather_kernel(table_hbm, idx_hbm, out_hbm, idx_v, *scratch):
    bufs = scratch[:_NCHUNK]
    rsems = scratch[_NCHUNK:2 * _NCHUNK]
    wsems = scratch[2 * _NCHUNK:]
    wid = lax.axis_index("s") * _NC + lax.axis_index("c")
    base = wid * _ROWS_PER_W
    pltpu.sync_copy(idx_hbm.at[base // T, pl.ds(base % T, _ROWS_PER_W)], idx_v)
    reads = [
        pltpu.async_copy(
            table_hbm.at[idx_v.at[pl.ds(k * _CHUNK, _CHUNK)]],
            bufs[k], rsems[k])
        for k in range(_NCHUNK)
    ]
    writes = []
    for k in range(_NCHUNK):
        reads[k].wait()
        writes.append(pltpu.async_copy(
            bufs[k], out_hbm.at[pl.ds(base + k * _CHUNK, _CHUNK)], wsems[k]))
    for w in writes:
        w.wait()


def _gather(table, codes):
    mesh = plsc.VectorSubcoreMesh(core_axis_name="c", subcore_axis_name="s")
    return pl.kernel(
        _gather_kernel,
        mesh=mesh,
        out_type=jax.ShapeDtypeStruct((TOK_SC, HALF), jnp.int32),
        scratch_types=(
            [pltpu.VMEM((_ROWS_PER_W,), jnp.int32)]
            + [pltpu.VMEM((_CHUNK, HALF), jnp.int32)] * _NCHUNK
            + [pltpu.SemaphoreType.DMA] * (2 * _NCHUNK)
        ),
    )(table, codes)


_OUT_SHAPE = jax.ShapeDtypeStruct((B, OUT_DIM, T), jnp.float32)


_T_TILE = 512
_NT = T // _T_TILE


def _onehot_body(codes_ref, p_ref, o_ref):
    b = pl.program_id(0) + B_SC
    iota = lax.broadcasted_iota(jnp.int32, (K, _T_TILE), 0)
    for t in range(_NT):
        cb = codes_ref[b, pl.ds(t * _T_TILE, _T_TILE)]
        oh = (iota == cb[None, :]).astype(jnp.bfloat16)
        o_ref[0, :, pl.ds(t * _T_TILE, _T_TILE)] = lax.dot_general(
            p_ref[...], oh,
            dimension_numbers=(((0,), (0,)), ((), ())),
            preferred_element_type=jnp.float32,
        )


def _onehot(codes, p16):
    return pl.pallas_call(
        _onehot_body,
        grid=(B_TC,),
        in_specs=[
            pl.BlockSpec((B, T), lambda i: (0, 0)),
            pl.BlockSpec((K, OUT_DIM), lambda i: (0, 0)),
        ],
        out_specs=pl.BlockSpec((1, OUT_DIM, T), lambda i: (i + B_SC, 0, 0)),
        out_shape=_OUT_SHAPE,
        compiler_params=pltpu.CompilerParams(
            dimension_semantics=("arbitrary",)),
    )(codes, p16)


def _unpack(words):
    lo16 = ((words << 16) >> 16).astype(jnp.int16)
    return lax.bitcast_convert_type(lo16, jnp.bfloat16).astype(jnp.float32)


def _tpose_body(prev_ref, g_ref, o_ref):
    del prev_ref
    gt = g_ref[...].T
    o_ref[0, :HALF] = _unpack(gt)
    o_ref[0, HALF:] = _unpack(gt >> 16)


def _transpose(prev, g):
    return pl.pallas_call(
        _tpose_body,
        grid=(B_SC,),
        in_specs=[
            pl.BlockSpec(memory_space=pl.ANY),
            pl.BlockSpec((T, HALF), lambda i: (i, 0)),
        ],
        out_specs=pl.BlockSpec((1, OUT_DIM, T), lambda i: (i, 0, 0)),
        out_shape=_OUT_SHAPE,
        input_output_aliases={0: 0},
        compiler_params=pltpu.CompilerParams(
            dimension_semantics=("parallel",)),
    )(prev, g)


def kernel(c, centroids, W, b):
    packed, p16 = _project(centroids, W, b)
    codes = c.astype(jnp.int32)
    g = _gather(packed, codes)
    out = _onehot(codes, p16)
    return _transpose(out, g)


# speedup vs baseline: 1.5307x; 1.0333x over previous
import functools

import jax
import jax.numpy as jnp
from jax import lax
from jax.experimental import pallas as pl
from jax.experimental.pallas import tpu as pltpu
from jax.experimental.pallas import tpu_sc as plsc

B, T = 8, 2048
K, CDIM, OUT_DIM = 1024, 1024, 768
N_TOK = B * T
HALF = OUT_DIM // 2
B_SC = 4
B_TC = B - B_SC
TOK_SC = B_SC * T

_K_BLK = K // 2


def _proj_body(cent_ref, w_ref, b_ref, packed_ref, p16_ref):
    p = lax.dot_general(
        cent_ref[...].astype(jnp.bfloat16), w_ref[...].astype(jnp.bfloat16),
        dimension_numbers=(((1,), (1,)), ((), ())),
        preferred_element_type=jnp.float32,
    ) + b_ref[...]
    pb = p.astype(jnp.bfloat16)
    lo = lax.bitcast_convert_type(pb[:, :HALF], jnp.int16)
    hi = lax.bitcast_convert_type(pb[:, HALF:], jnp.int16)
    packed_ref[...] = (lo.astype(jnp.int32) & 0xFFFF) | (hi.astype(jnp.int32) << 16)
    p16_ref[...] = pb


def _project(centroids, W, b):
    return pl.pallas_call(
        _proj_body,
        grid=(K // _K_BLK,),
        in_specs=[
            pl.BlockSpec((_K_BLK, CDIM), lambda i: (i, 0)),
            pl.BlockSpec((OUT_DIM, CDIM), lambda i: (0, 0)),
            pl.BlockSpec((OUT_DIM,), lambda i: (0,)),
        ],
        out_specs=[
            pl.BlockSpec((_K_BLK, HALF), lambda i: (i, 0)),
            pl.BlockSpec((_K_BLK, OUT_DIM), lambda i: (i, 0)),
        ],
        out_shape=[
            jax.ShapeDtypeStruct((K, HALF), jnp.int32),
            jax.ShapeDtypeStruct((K, OUT_DIM), jnp.bfloat16),
        ],
        compiler_params=pltpu.CompilerParams(
            dimension_semantics=("parallel",)),
    )(centroids, W, b)


_NC, _NS = 2, 16
_NW = _NC * _NS
_ROWS_PER_W = TOK_SC // _NW
_CHUNK = 64
_NCHUNK = _ROWS_PER_W // _CHUNK


def _gather_kernel(table_hbm, idx_hbm, out_hbm, idx_v, *scratch):
    bufs = scratch[:_NCHUNK]
    rsems = scratch[_NCHUNK:2 * _NCHUNK]
    wsems = scratch[2 * _NCHUNK:]
    wid = lax.axis_index("s") * _NC + lax.axis_index("c")
    base = wid * _ROWS_PER_W
    pltpu.sync_copy(idx_hbm.at[base // T, pl.ds(base % T, _ROWS_PER_W)], idx_v)
    reads = [
        pltpu.async_copy(
            table_hbm.at[idx_v.at[pl.ds(k * _CHUNK, _CHUNK)]],
            bufs[k], rsems[k])
        for k in range(_NCHUNK)
    ]
    writes = []
    for k in range(_NCHUNK):
        reads[k].wait()
        writes.append(pltpu.async_copy(
            bufs[k], out_hbm.at[pl.ds(base + k * _CHUNK, _CHUNK)], wsems[k]))
    for w in writes:
        w.wait()


def _gather(table, codes):
    mesh = plsc.VectorSubcoreMesh(core_axis_name="c", subcore_axis_name="s")
    return pl.kernel(
        _gather_kernel,
        mesh=mesh,
        out_type=jax.ShapeDtypeStruct((TOK_SC, HALF), jnp.int32),
        scratch_types=(
            [pltpu.VMEM((_ROWS_PER_W,), jnp.int32)]
            + [pltpu.VMEM((_CHUNK, HALF), jnp.int32)] * _NCHUNK
            + [pltpu.SemaphoreType.DMA] * (2 * _NCHUNK)
        ),
    )(table, codes)


_OUT_SHAPE = jax.ShapeDtypeStruct((B, OUT_DIM, T), jnp.float32)


_T_TILE = 256
_NT = T // _T_TILE


def _onehot_body(codes_ref, p_ref, o_ref):
    b = pl.program_id(0) + B_SC
    iota = lax.broadcasted_iota(jnp.int32, (K, _T_TILE), 0)
    for t in range(_NT):
        cb = codes_ref[b, pl.ds(t * _T_TILE, _T_TILE)]
        oh = (iota == cb[None, :]).astype(jnp.bfloat16)
        o_ref[0, :, pl.ds(t * _T_TILE, _T_TILE)] = lax.dot_general(
            p_ref[...], oh,
            dimension_numbers=(((0,), (0,)), ((), ())),
            preferred_element_type=jnp.float32,
        )


def _onehot(codes, p16):
    return pl.pallas_call(
        _onehot_body,
        grid=(B_TC,),
        in_specs=[
            pl.BlockSpec((B, T), lambda i: (0, 0)),
            pl.BlockSpec((K, OUT_DIM), lambda i: (0, 0)),
        ],
        out_specs=pl.BlockSpec((1, OUT_DIM, T), lambda i: (i + B_SC, 0, 0)),
        out_shape=_OUT_SHAPE,
        compiler_params=pltpu.CompilerParams(
            dimension_semantics=("arbitrary",)),
    )(codes, p16)


def _unpack(words):
    lo16 = ((words << 16) >> 16).astype(jnp.int16)
    return lax.bitcast_convert_type(lo16, jnp.bfloat16).astype(jnp.float32)


def _tpose_body(prev_ref, g_ref, o_ref):
    del prev_ref
    gt = g_ref[...].T
    o_ref[0, :HALF] = _unpack(gt)
    o_ref[0, HALF:] = _unpack(gt >> 16)


def _transpose(prev, g):
    return pl.pallas_call(
        _tpose_body,
        grid=(B_SC,),
        in_specs=[
            pl.BlockSpec(memory_space=pl.ANY),
            pl.BlockSpec((T, HALF), lambda i: (i, 0)),
        ],
        out_specs=pl.BlockSpec((1, OUT_DIM, T), lambda i: (i, 0, 0)),
        out_shape=_OUT_SHAPE,
        input_output_aliases={0: 0},
        compiler_params=pltpu.CompilerParams(
            dimension_semantics=("parallel",)),
    )(prev, g)


def kernel(c, centroids, W, b):
    packed, p16 = _project(centroids, W, b)
    codes = c.astype(jnp.int32)
    g = _gather(packed, codes)
    out = _onehot(codes, p16)
    return _transpose(out, g)
